# baseline (speedup 1.0000x reference)
import functools

import jax
import jax.numpy as jnp
from jax import lax
from jax.experimental import pallas as pl
from jax.experimental.pallas import tpu as pltpu
from jax.experimental.pallas import tpu_sc as plsc

D = 128
S = 200
L = 16

_RSQRT_MAGIC = 0x5F3759DF


def _rsqrt(v):
    bits = plsc.bitcast(v, jnp.int32)
    y = plsc.bitcast(jnp.int32(_RSQRT_MAGIC) - jnp.right_shift(bits, 1),
                     jnp.float32)
    for _ in range(3):
        y = y * (1.5 - 0.5 * v * y * y)
    return y


def _make_kernel(n_tok, vocab, tt_vocab, max_pos):
    info = plsc.get_sparse_core_info()
    nw = info.num_cores * info.num_subcores
    per_w = n_tok // nw
    chunk = 320
    n_chunk = per_w // chunk
    n_grp = chunk // L

    mesh = plsc.VectorSubcoreMesh(core_axis_name="c", subcore_axis_name="s")

    @functools.partial(
        pl.kernel,
        out_type=jax.ShapeDtypeStruct((n_tok, D), jnp.float32),
        mesh=mesh,
        compiler_params=pltpu.CompilerParams(needs_layout_passes=False),
        scratch_types=[
            pltpu.VMEM((chunk,), jnp.int32),
            pltpu.VMEM((chunk,), jnp.int32),
            pltpu.VMEM((chunk, D), jnp.float32),
            pltpu.VMEM((S, D), jnp.float32),
            pltpu.VMEM((16, D), jnp.float32),
            pltpu.VMEM((D,), jnp.float32),
            pltpu.VMEM((D,), jnp.float32),
            pltpu.SemaphoreType.DMA,
        ],
    )
    def run(ids_hbm, tts_hbm, wtab_hbm, tttab_hbm, pos_hbm, gam_hbm, bet_hbm,
            out_hbm, wid_v, tid_v, buf, pos_v, tt_v, gam_v, bet_v, sem):
        wid = lax.axis_index("s") * info.num_cores + lax.axis_index("c")
        base_w = wid * per_w

        pltpu.sync_copy(pos_hbm.at[pl.ds(0, S)], pos_v)
        pltpu.sync_copy(tttab_hbm, tt_v)
        pltpu.sync_copy(gam_hbm, gam_v)
        pltpu.sync_copy(bet_hbm, bet_v)

        iota = lax.iota(jnp.int32, L)
        inv_d = jnp.float32(1.0 / D)
        eps = jnp.float32(1e-12)

        def do_chunk(c, carry):
            base = base_w + c * chunk
            pltpu.sync_copy(ids_hbm.at[pl.ds(base, chunk)], wid_v)
            pltpu.sync_copy(tts_hbm.at[pl.ds(base, chunk)], tid_v)

            cps = []
            for off, n in ((0, 128), (128, 128), (256, 64)):
                cps.append(pltpu.async_copy(
                    wtab_hbm.at[wid_v.at[pl.ds(off, n)]],
                    buf.at[pl.ds(off, n)], sem))
            for cp in cps:
                cp.wait()

            def do_group(g, gcarry):
                t0 = g * L
                tok = t0 + iota
                wids = wid_v[pl.ds(t0, L)]
                tids = tid_v[pl.ds(t0, L)]
                wm = wids != 0
                tm = tids != 0
                pidx = jnp.mod(c * chunk + t0 + iota, S)
                zero = jnp.zeros((L,), jnp.float32)

                def col_sum(j, sums):
                    s1, s2 = sums
                    jv = jnp.full((L,), j, jnp.int32)
                    w = plsc.load_gather(buf, [tok, jv])
                    t = plsc.load_gather(tt_v, [tids, jv])
                    p = plsc.load_gather(pos_v, [pidx, jv])
                    x = jnp.where(wm, w, zero) + jnp.where(tm, t, zero) + p
                    plsc.store_scatter(buf, [tok, jv], x)
                    return (s1 + x, s2 + x * x)

                s1, s2 = lax.fori_loop(0, D, col_sum, (zero, zero))
                mean = s1 * inv_d
                var = s2 * inv_d - mean * mean
                inv = _rsqrt(var + eps)
                minv = mean * inv

                def col_norm(j, dummy):
                    jv = jnp.full((L,), j, jnp.int32)
                    x = plsc.load_gather(buf, [tok, jv])
                    ga = plsc.load_gather(gam_v, [jv])
                    be = plsc.load_gather(bet_v, [jv])
                    y = (x * inv - minv) * ga + be
                    plsc.store_scatter(buf, [tok, jv], y)
                    return dummy

                lax.fori_loop(0, D, col_norm, 0)
                return gcarry

            lax.fori_loop(0, n_grp, do_group, 0)
            pltpu.sync_copy(buf, out_hbm.at[pl.ds(base, chunk)])
            return carry

        lax.fori_loop(0, n_chunk, do_chunk, 0)

    return run


def kernel(input_ids, token_type_ids, word_table, tt_table, pos_emb,
           ln_gamma, ln_beta):
    b, s = input_ids.shape
    n_tok = b * s
    run = _make_kernel(n_tok, word_table.shape[0], tt_table.shape[0],
                       pos_emb.shape[0])
    out = run(input_ids.reshape(n_tok).astype(jnp.int32),
              token_type_ids.reshape(n_tok).astype(jnp.int32),
              word_table, tt_table, pos_emb, ln_gamma, ln_beta)
    return out.reshape(b, s, D)


# speedup vs baseline: 1.9932x; 1.9932x over previous
import functools

import jax
import jax.numpy as jnp
from jax import lax
from jax.experimental import pallas as pl
from jax.experimental.pallas import tpu as pltpu
from jax.experimental.pallas import tpu_sc as plsc

D = 128
S = 200
L = 16

_RSQRT_MAGIC = 0x5F3759DF


def _rsqrt(v):
    bits = plsc.bitcast(v, jnp.int32)
    y = plsc.bitcast(jnp.int32(_RSQRT_MAGIC) - jnp.right_shift(bits, 1),
                     jnp.float32)
    for _ in range(3):
        y = y * (1.5 - 0.5 * v * y * y)
    return y


def _make_kernel(n_tok, vocab, tt_vocab, max_pos):
    info = plsc.get_sparse_core_info()
    nw = info.num_cores * info.num_subcores
    per_w = n_tok // nw
    chunk = 320
    n_chunk = per_w // chunk
    n_grp = chunk // L

    mesh = plsc.VectorSubcoreMesh(core_axis_name="c", subcore_axis_name="s")

    @functools.partial(
        pl.kernel,
        out_type=jax.ShapeDtypeStruct((n_tok, D), jnp.float32),
        mesh=mesh,
        compiler_params=pltpu.CompilerParams(needs_layout_passes=False),
        scratch_types=[
            pltpu.VMEM((chunk,), jnp.int32),
            pltpu.VMEM((chunk,), jnp.int32),
            pltpu.VMEM((chunk, D), jnp.float32),
            pltpu.VMEM((D, L), jnp.float32),
            pltpu.VMEM((S, D), jnp.float32),
            pltpu.VMEM((16, D), jnp.float32),
            pltpu.VMEM((D,), jnp.float32),
            pltpu.VMEM((D,), jnp.float32),
            pltpu.SemaphoreType.DMA,
        ],
    )
    def run(ids_hbm, tts_hbm, wtab_hbm, tttab_hbm, pos_hbm, gam_hbm, bet_hbm,
            out_hbm, wid_v, tid_v, buf, xt_v, pos_v, tt_v, gam_v, bet_v, sem):
        wid = lax.axis_index("s") * info.num_cores + lax.axis_index("c")
        base_w = wid * per_w

        pltpu.sync_copy(pos_hbm.at[pl.ds(0, S)], pos_v)
        pltpu.sync_copy(tttab_hbm, tt_v)
        pltpu.sync_copy(gam_hbm, gam_v)
        pltpu.sync_copy(bet_hbm, bet_v)

        iota = lax.iota(jnp.int32, L)
        inv_d = jnp.float32(1.0 / D)
        eps = jnp.float32(1e-12)

        def do_chunk(c, carry):
            base = base_w + c * chunk
            pltpu.sync_copy(ids_hbm.at[pl.ds(base, chunk)], wid_v)
            pltpu.sync_copy(tts_hbm.at[pl.ds(base, chunk)], tid_v)

            cps = []
            for off, n in ((0, 128), (128, 128), (256, 64)):
                cps.append(pltpu.async_copy(
                    wtab_hbm.at[wid_v.at[pl.ds(off, n)]],
                    buf.at[pl.ds(off, n)], sem))
            for cp in cps:
                cp.wait()

            def do_group(g, gcarry):
                t0 = g * L
                tok = t0 + iota
                wids = wid_v[pl.ds(t0, L)]
                tids = tid_v[pl.ds(t0, L)]
                wm = wids != 0
                tm = tids != 0
                pidx = jnp.mod(c * chunk + t0 + iota, S)
                zero = jnp.zeros((L,), jnp.float32)

                @plsc.parallel_loop(0, D, unroll=8, carry=(zero, zero))
                def col_sum(j, sums):
                    s1, s2 = sums
                    jv = jnp.full((L,), j, jnp.int32)
                    w = plsc.load_gather(buf, [tok, jv])
                    t = plsc.load_gather(tt_v, [tids, jv])
                    p = plsc.load_gather(pos_v, [pidx, jv])
                    x = jnp.where(wm, w, zero) + jnp.where(tm, t, zero) + p
                    xt_v[j] = x
                    return (s1 + x, s2 + x * x)

                s1, s2 = col_sum
                mean = s1 * inv_d
                var = s2 * inv_d - mean * mean
                inv = _rsqrt(var + eps)
                minv = mean * inv

                @plsc.parallel_loop(0, D, unroll=8)
                def col_norm(j):
                    jv = jnp.full((L,), j, jnp.int32)
                    x = xt_v[j]
                    ga = plsc.load_gather(gam_v, [jv])
                    be = plsc.load_gather(bet_v, [jv])
                    y = (x * inv - minv) * ga + be
                    plsc.store_scatter(buf, [tok, jv], y)

                return gcarry

            lax.fori_loop(0, n_grp, do_group, 0)
            pltpu.sync_copy(buf, out_hbm.at[pl.ds(base, chunk)])
            return carry

        lax.fori_loop(0, n_chunk, do_chunk, 0)

    return run


def kernel(input_ids, token_type_ids, word_table, tt_table, pos_emb,
           ln_gamma, ln_beta):
    b, s = input_ids.shape
    n_tok = b * s
    run = _make_kernel(n_tok, word_table.shape[0], tt_table.shape[0],
                       pos_emb.shape[0])
    out = run(input_ids.reshape(n_tok).astype(jnp.int32),
              token_type_ids.reshape(n_tok).astype(jnp.int32),
              word_table, tt_table, pos_emb, ln_gamma, ln_beta)
    return out.reshape(b, s, D)


# speedup vs baseline: 2.6274x; 1.3182x over previous
import functools

import jax
import jax.numpy as jnp
from jax import lax
from jax.experimental import pallas as pl
from jax.experimental.pallas import tpu as pltpu
from jax.experimental.pallas import tpu_sc as plsc

D = 128
L = 16

_RSQRT_MAGIC = 0x5F3759DF


def _rsqrt(v):
    bits = plsc.bitcast(v, jnp.int32)
    y = plsc.bitcast(jnp.int32(_RSQRT_MAGIC) - jnp.right_shift(bits, 1),
                     jnp.float32)
    for _ in range(3):
        y = y * (1.5 - 0.5 * v * y * y)
    return y


def _make_kernel(batch, seq):
    info = plsc.get_sparse_core_info()
    nw = info.num_cores * info.num_subcores
    n_tok = batch * seq
    ch = 256
    cb_per_s = batch // ch
    per_w = (n_tok // ch) // nw
    n_grp = ch // L

    mesh = plsc.VectorSubcoreMesh(core_axis_name="c", subcore_axis_name="s")

    @functools.partial(
        pl.kernel,
        out_type=jax.ShapeDtypeStruct((n_tok, D), jnp.float32),
        mesh=mesh,
        compiler_params=pltpu.CompilerParams(needs_layout_passes=False),
        scratch_types=[
            pltpu.VMEM((ch,), jnp.int32),
            pltpu.VMEM((ch,), jnp.int32),
            pltpu.VMEM((ch,), jnp.int32),
            pltpu.VMEM((ch,), jnp.int32),
            pltpu.VMEM((ch, D), jnp.float32),
            pltpu.VMEM((ch, D), jnp.float32),
            pltpu.VMEM((D,), jnp.int32),
            pltpu.VMEM((D,), jnp.int32),
            pltpu.VMEM((D,), jnp.int32),
            pltpu.VMEM((D,), jnp.int32),
            pltpu.VMEM((D, L), jnp.float32),
            pltpu.VMEM((16, D), jnp.float32),
            pltpu.VMEM((16, D), jnp.float32),
            pltpu.VMEM((1, D), jnp.float32),
            pltpu.VMEM((D,), jnp.float32),
            pltpu.VMEM((D,), jnp.float32),
            pltpu.SemaphoreType.DMA,
            pltpu.SemaphoreType.DMA,
            pltpu.SemaphoreType.DMA,
            pltpu.SemaphoreType.DMA,
        ],
    )
    def run(ids_hbm, tts_hbm, wtab_hbm, tttab_hbm, pos_hbm, gam_hbm, bet_hbm,
            out_hbm, wid0, wid1, tid0, tid1, buf0, buf1,
            oi0a, oi0b, oi1a, oi1b, xt_v, tt0_v, tte_v, prow_v,
            gam_v, bet_v, gsem0, gsem1, ssem0, ssem1):
        wid_b = (wid0, wid1)
        tid_b = (tid0, tid1)
        buf_b = (buf0, buf1)
        oi_b = ((oi0a, oi0b), (oi1a, oi1b))
        gsem_b = (gsem0, gsem1)
        ssem_b = (ssem0, ssem1)

        w = lax.axis_index("s") * info.num_cores + lax.axis_index("c")
        cid0 = w * per_w

        pltpu.sync_copy(tttab_hbm, tt0_v)
        pltpu.sync_copy(gam_hbm, gam_v)
        pltpu.sync_copy(bet_hbm, bet_v)
        zero16 = jnp.zeros((L,), jnp.float32)
        for k in range(D // L):
            tt0_v[0, pl.ds(k * L, L)] = zero16

        iota = lax.iota(jnp.int32, L)
        inv_d = jnp.float32(1.0 / D)
        eps = jnp.float32(1e-12)

        def chunk_pos(c):
            cid = cid0 + c
            s = cid // cb_per_s
            b0 = (cid - s * cb_per_s) * ch
            return s, b0

        def copy_ids(c, b):
            s, b0 = chunk_pos(c)
            base = s * batch + b0
            pltpu.sync_copy(ids_hbm.at[pl.ds(base, ch)], wid_b[b])
            pltpu.sync_copy(tts_hbm.at[pl.ds(base, ch)], tid_b[b])

        def gather_start(b):
            for h in range(2):
                pltpu.async_copy(
                    wtab_hbm.at[wid_b[b].at[pl.ds(h * D, D)]],
                    buf_b[b].at[pl.ds(h * D, D)], gsem_b[b])

        def gather_wait(b):
            for h in range(2):
                pltpu.make_async_copy(
                    wtab_hbm.at[wid_b[b].at[pl.ds(h * D, D)]],
                    buf_b[b].at[pl.ds(h * D, D)], gsem_b[b]).wait()

        def scatter_start(b):
            for h in range(2):
                pltpu.async_copy(
                    buf_b[b].at[pl.ds(h * D, D)],
                    out_hbm.at[oi_b[b][h]], ssem_b[b])

        def scatter_wait(b):
            for h in range(2):
                pltpu.make_async_copy(
                    buf_b[b].at[pl.ds(h * D, D)],
                    out_hbm.at[oi_b[b][h]], ssem_b[b]).wait()

        def compute(c, b):
            s, b0 = chunk_pos(c)
            buf = buf_b[b]

            pltpu.sync_copy(pos_hbm.at[pl.ds(s, 1)], prow_v)
            for r in range(16):
                for k in range(D // L):
                    sl = pl.ds(k * L, L)
                    tte_v[r, sl] = tt0_v[r, sl] + prow_v[0, sl]

            def do_group(g, gcarry):
                t0 = g * L
                tok = t0 + iota
                wids = wid_b[b][pl.ds(t0, L)]
                tids = tid_b[b][pl.ds(t0, L)]
                wm = wids != 0

                @plsc.parallel_loop(0, D, unroll=8, carry=(zero16, zero16))
                def col_sum(j, sums):
                    s1, s2 = sums
                    jv = jnp.full((L,), j, jnp.int32)
                    wv = plsc.load_gather(buf, [tok, jv])
                    te = plsc.load_gather(tte_v, [tids, jv])
                    x = jnp.where(wm, wv, zero16) + te
                    xt_v[j] = x
                    return (s1 + x, s2 + x * x)

                s1, s2 = col_sum
                mean = s1 * inv_d
                var = s2 * inv_d - mean * mean
                inv = _rsqrt(var + eps)
                minv = mean * inv

                @plsc.parallel_loop(0, D, unroll=8)
                def col_norm(j):
                    jv = jnp.full((L,), j, jnp.int32)
                    x = xt_v[j]
                    ga = plsc.load_gather(gam_v, [jv])
                    be = plsc.load_gather(bet_v, [jv])
                    y = (x * inv - minv) * ga + be
                    plsc.store_scatter(buf, [tok, jv], y)

                return gcarry

            lax.fori_loop(0, n_grp, do_group, 0)

            obase = b0 * seq + s
            for h in range(2):
                oref = oi_b[b][h]
                for k in range(D // L):
                    oref[pl.ds(k * L, L)] = (
                        obase + (h * D + k * L + iota) * seq)

        copy_ids(0, 0)
        gather_start(0)

        def pair(p, carry):
            cc = p * 2
            for db in range(2):
                c = cc + db

                @pl.when(c < per_w)
                def _():
                    nxt = 1 - db
                    @pl.when(c + 1 < per_w)
                    def _():
                        copy_ids(c + 1, nxt)
                        @pl.when(c >= 1)
                        def _():
                            scatter_wait(nxt)
                        gather_start(nxt)
                    gather_wait(db)
                    compute(c, db)
                    scatter_start(db)
            return carry

        lax.fori_loop(0, (per_w + 1) // 2, pair, 0)
        scatter_wait((per_w - 1) % 2)
        scatter_wait(per_w % 2)

    return run


def kernel(input_ids, token_type_ids, word_table, tt_table, pos_emb,
           ln_gamma, ln_beta):
    b, s = input_ids.shape
    run = _make_kernel(b, s)
    out = run(input_ids.T.reshape(-1).astype(jnp.int32),
              token_type_ids.T.reshape(-1).astype(jnp.int32),
              word_table, tt_table, pos_emb, ln_gamma, ln_beta)
    return out.reshape(b, s, D)


# speedup vs baseline: 12.0824x; 4.5986x over previous
import functools

import jax
import jax.numpy as jnp
from jax import lax
from jax.experimental import pallas as pl
from jax.experimental.pallas import tpu as pltpu
from jax.experimental.pallas import tpu_sc as plsc

D = 128
L = 16

_RSQRT_MAGIC = 0x5F3759DF


def _rsqrt(v):
    bits = plsc.bitcast(v, jnp.int32)
    y = plsc.bitcast(jnp.int32(_RSQRT_MAGIC) - jnp.right_shift(bits, 1),
                     jnp.float32)
    for _ in range(3):
        y = y * (1.5 - 0.5 * v * y * y)
    return y


def _make_kernel(batch, seq):
    info = plsc.get_sparse_core_info()
    nw = info.num_cores * info.num_subcores
    n_tok = batch * seq
    ch = 256
    cb_per_s = batch // ch
    per_w = (n_tok // ch) // nw
    n_grp = ch // L

    mesh = plsc.VectorSubcoreMesh(core_axis_name="c", subcore_axis_name="s")

    @functools.partial(
        pl.kernel,
        out_type=jax.ShapeDtypeStruct((n_tok, D), jnp.float32),
        mesh=mesh,
        compiler_params=pltpu.CompilerParams(needs_layout_passes=False),
        scratch_types=[
            pltpu.VMEM((ch,), jnp.int32),
            pltpu.VMEM((ch,), jnp.int32),
            pltpu.VMEM((ch,), jnp.int32),
            pltpu.VMEM((ch,), jnp.int32),
            pltpu.VMEM((ch, D), jnp.float32),
            pltpu.VMEM((ch, D), jnp.float32),
            pltpu.VMEM((D,), jnp.int32),
            pltpu.VMEM((D,), jnp.int32),
            pltpu.VMEM((D,), jnp.int32),
            pltpu.VMEM((D,), jnp.int32),
            pltpu.VMEM((D, L), jnp.float32),
            pltpu.VMEM((16, D), jnp.float32),
            pltpu.VMEM((16, D), jnp.float32),
            pltpu.VMEM((1, D), jnp.float32),
            pltpu.VMEM((D,), jnp.float32),
            pltpu.VMEM((D,), jnp.float32),
            pltpu.SemaphoreType.DMA,
            pltpu.SemaphoreType.DMA,
            pltpu.SemaphoreType.DMA,
            pltpu.SemaphoreType.DMA,
        ],
    )
    def run(ids_hbm, tts_hbm, wtab_hbm, tttab_hbm, pos_hbm, gam_hbm, bet_hbm,
            out_hbm, wid0, wid1, tid0, tid1, buf0, buf1,
            oi0a, oi0b, oi1a, oi1b, xt_v, tt0_v, tte_v, prow_v,
            gam_v, bet_v, gsem0, gsem1, ssem0, ssem1):
        wid_b = (wid0, wid1)
        tid_b = (tid0, tid1)
        buf_b = (buf0, buf1)
        oi_b = ((oi0a, oi0b), (oi1a, oi1b))
        gsem_b = (gsem0, gsem1)
        ssem_b = (ssem0, ssem1)

        w = lax.axis_index("s") * info.num_cores + lax.axis_index("c")
        cid0 = w * per_w

        pltpu.sync_copy(tttab_hbm, tt0_v)
        pltpu.sync_copy(gam_hbm, gam_v)
        pltpu.sync_copy(bet_hbm, bet_v)
        zero16 = jnp.zeros((L,), jnp.float32)
        for k in range(D // L):
            tt0_v[0, pl.ds(k * L, L)] = zero16

        iota = lax.iota(jnp.int32, L)
        inv_d = jnp.float32(1.0 / D)
        eps = jnp.float32(1e-12)

        def chunk_pos(c):
            cid = cid0 + c
            s = cid // cb_per_s
            b0 = (cid - s * cb_per_s) * ch
            return s, b0

        def copy_ids(c, b):
            s, b0 = chunk_pos(c)
            base = s * batch + b0
            pltpu.sync_copy(ids_hbm.at[pl.ds(base, ch)], wid_b[b])
            pltpu.sync_copy(tts_hbm.at[pl.ds(base, ch)], tid_b[b])

        def gather_start(b):
            for h in range(2):
                pltpu.async_copy(
                    wtab_hbm.at[wid_b[b].at[pl.ds(h * D, D)]],
                    buf_b[b].at[pl.ds(h * D, D)], gsem_b[b])

        def gather_wait(b):
            for h in range(2):
                pltpu.make_async_copy(
                    wtab_hbm.at[wid_b[b].at[pl.ds(h * D, D)]],
                    buf_b[b].at[pl.ds(h * D, D)], gsem_b[b]).wait()

        def scatter_start(b):
            for h in range(2):
                pltpu.async_copy(
                    buf_b[b].at[pl.ds(h * D, D)],
                    out_hbm.at[oi_b[b][h]], ssem_b[b])

        def scatter_wait(b):
            for h in range(2):
                pltpu.make_async_copy(
                    buf_b[b].at[pl.ds(h * D, D)],
                    out_hbm.at[oi_b[b][h]], ssem_b[b]).wait()

        def compute(c, b):
            s, b0 = chunk_pos(c)
            buf = buf_b[b]

            pltpu.sync_copy(pos_hbm.at[pl.ds(s, 1)], prow_v)
            for r in range(16):
                for k in range(D // L):
                    sl = pl.ds(k * L, L)
                    tte_v[r, sl] = tt0_v[r, sl] + prow_v[0, sl]

            def do_group(g, gcarry):
                t0 = g * L
                tok = t0 + iota
                wids = wid_b[b][pl.ds(t0, L)]
                tids = tid_b[b][pl.ds(t0, L)]
                wm = wids != 0

                @plsc.parallel_loop(0, D, unroll=8, carry=(zero16, zero16))
                def col_sum(j, sums):
                    s1, s2 = sums
                    rv = (jnp.full((L,), j, jnp.int32) + iota) & (D - 1)
                    wv = plsc.load_gather(buf, [tok, rv])
                    te = plsc.load_gather(tte_v, [tids, rv])
                    x = jnp.where(wm, wv, zero16) + te
                    xt_v[j] = x
                    return (s1 + x, s2 + x * x)

                s1, s2 = col_sum
                mean = s1 * inv_d
                var = s2 * inv_d - mean * mean
                inv = _rsqrt(var + eps)
                minv = mean * inv

                @plsc.parallel_loop(0, D, unroll=8)
                def col_norm(j):
                    rv = (jnp.full((L,), j, jnp.int32) + iota) & (D - 1)
                    x = xt_v[j]
                    ga = plsc.load_gather(gam_v, [rv])
                    be = plsc.load_gather(bet_v, [rv])
                    y = (x * inv - minv) * ga + be
                    plsc.store_scatter(buf, [tok, rv], y)

                return gcarry

            lax.fori_loop(0, n_grp, do_group, 0)

            obase = b0 * seq + s
            for h in range(2):
                oref = oi_b[b][h]
                for k in range(D // L):
                    oref[pl.ds(k * L, L)] = (
                        obase + (h * D + k * L + iota) * seq)

        copy_ids(0, 0)
        gather_start(0)

        def pair(p, carry):
            cc = p * 2
            for db in range(2):
                c = cc + db

                @pl.when(c < per_w)
                def _():
                    nxt = 1 - db
                    @pl.when(c + 1 < per_w)
                    def _():
                        copy_ids(c + 1, nxt)
                        @pl.when(c >= 1)
                        def _():
                            scatter_wait(nxt)
                        gather_start(nxt)
                    gather_wait(db)
                    compute(c, db)
                    scatter_start(db)
            return carry

        lax.fori_loop(0, (per_w + 1) // 2, pair, 0)
        scatter_wait((per_w - 1) % 2)
        scatter_wait(per_w % 2)

    return run


def kernel(input_ids, token_type_ids, word_table, tt_table, pos_emb,
           ln_gamma, ln_beta):
    b, s = input_ids.shape
    run = _make_kernel(b, s)
    out = run(input_ids.T.reshape(-1).astype(jnp.int32),
              token_type_ids.T.reshape(-1).astype(jnp.int32),
              word_table, tt_table, pos_emb, ln_gamma, ln_beta)
    return out.reshape(b, s, D)
